# baseline (speedup 1.0000x reference)
import functools

import jax
import jax.numpy as jnp
from jax import lax
from jax.experimental import pallas as pl
from jax.experimental.pallas import tpu as pltpu
from jax.experimental.pallas import tpu_sc as plsc

B, L = 4096, 50
EMBED = 800
LANES = 16
NC, NS = 2, 16
NW = NC * NS
RPW = B // NW
LPAD = 56
GRP = 16


def _pool(x_pad, emb):
    mesh = plsc.VectorSubcoreMesh(
        core_axis_name="c", subcore_axis_name="s", num_cores=NC, num_subcores=NS
    )

    @functools.partial(
        pl.kernel,
        out_type=jax.ShapeDtypeStruct((B, EMBED), jnp.float32),
        mesh=mesh,
        scratch_types=[
            pltpu.VMEM((RPW, LPAD), jnp.int32),
            pltpu.VMEM((LPAD, EMBED), jnp.float32),
            pltpu.VMEM((LPAD, EMBED), jnp.float32),
            pltpu.VMEM((GRP, EMBED), jnp.float32),
            pltpu.SemaphoreType.DMA,
            pltpu.SemaphoreType.DMA,
        ],
        compiler_params=pltpu.CompilerParams(use_tc_tiling_on_sc=False),
    )
    def k(x_hbm, emb_hbm, out_hbm, idx_v, ga, gb, obuf, sema, semb):
        wid = lax.axis_index("s") * NC + lax.axis_index("c")
        base = wid * RPW
        pltpu.sync_copy(x_hbm.at[pl.ds(base, RPW)], idx_v)

        def start(r, buf, sem):
            pltpu.make_async_copy(emb_hbm.at[idx_v.at[r]], buf, sem).start()

        def accum(buf, orow):
            def dchunk(d, c):
                s = d * LANES
                accs = [jnp.zeros((LANES,), jnp.float32) for _ in range(5)]
                for j in range(L):
                    accs[j % 5] = accs[j % 5] + buf[j, pl.ds(s, LANES)]
                tot = (accs[0] + accs[1]) + (accs[2] + accs[3]) + accs[4]
                obuf[orow, pl.ds(s, LANES)] = tot * (1.0 / L)
                return c

            lax.fori_loop(0, EMBED // LANES, dchunk, 0, unroll=2)

        start(0, ga, sema)

        def group(g, carry):
            def pair(p, c0):
                r0 = g * GRP + 2 * p
                start(r0 + 1, gb, semb)
                pltpu.make_async_copy(emb_hbm.at[idx_v.at[r0]], ga, sema).wait()
                accum(ga, 2 * p)

                @pl.when(r0 + 2 < RPW)
                def _():
                    start(r0 + 2, ga, sema)

                pltpu.make_async_copy(emb_hbm.at[idx_v.at[r0]], gb, semb).wait()
                accum(gb, 2 * p + 1)
                return c0

            lax.fori_loop(0, GRP // 2, pair, 0)
            pltpu.sync_copy(obuf, out_hbm.at[pl.ds(base + g * GRP, GRP)])
            return carry

        lax.fori_loop(0, RPW // GRP, group, 0)

    return k(x_pad, emb)


BB = 512


def _mlp_body(p, w1, b1, w2, b2, w3, b3, w4, b4, w5, b5, o):
    h = p[...].astype(jnp.bfloat16)
    h = jnp.maximum(jnp.dot(h, w1[...], preferred_element_type=jnp.float32) + b1[...], 0.0)
    h = jnp.maximum(jnp.dot(h.astype(jnp.bfloat16), w2[...], preferred_element_type=jnp.float32) + b2[...], 0.0)
    h = jnp.maximum(jnp.dot(h.astype(jnp.bfloat16), w3[...], preferred_element_type=jnp.float32) + b3[...], 0.0)
    h = jnp.maximum(jnp.dot(h.astype(jnp.bfloat16), w4[...], preferred_element_type=jnp.float32) + b4[...], 0.0)
    o[...] = jnp.dot(h.astype(jnp.bfloat16), w5[...], preferred_element_type=jnp.float32) + b5[...]


def _mlp(pooled, w1, b1, w2, b2, w3, b3, w4, b4, w5p, b5p):
    full = lambda a: pl.BlockSpec(a.shape, lambda i: (0,) * a.ndim)
    return pl.pallas_call(
        _mlp_body,
        grid=(B // BB,),
        in_specs=[pl.BlockSpec((BB, EMBED), lambda i: (i, 0))]
        + [full(a) for a in (w1, b1, w2, b2, w3, b3, w4, b4, w5p, b5p)],
        out_specs=pl.BlockSpec((BB, 128), lambda i: (i, 0)),
        out_shape=jax.ShapeDtypeStruct((B, 128), jnp.float32),
        compiler_params=pltpu.CompilerParams(dimension_semantics=("arbitrary",)),
    )(pooled, w1, b1, w2, b2, w3, b3, w4, b4, w5p, b5p)


def kernel(x, emb, W1, b1, W2, b2, W3, b3, W4, b4, W5, b5):
    x_pad = jnp.pad(x, ((0, 0), (0, LPAD - L)))
    pooled = _pool(x_pad, emb)

    bf = jnp.bfloat16
    w5p = jnp.pad(W5, ((0, 0), (0, 128 - W5.shape[1])))
    b5p = jnp.pad(b5, ((0, 128 - b5.shape[0]),))
    out = _mlp(
        pooled,
        W1.astype(bf), b1.reshape(1, -1),
        W2.astype(bf), b2.reshape(1, -1),
        W3.astype(bf), b3.reshape(1, -1),
        W4.astype(bf), b4.reshape(1, -1),
        w5p.astype(bf), b5p.reshape(1, -1),
    )
    return out[:, : W5.shape[1]]


# speedup vs baseline: 5.3290x; 5.3290x over previous
import functools

import jax
import jax.numpy as jnp
from jax import lax
from jax.experimental import pallas as pl
from jax.experimental.pallas import tpu as pltpu
from jax.experimental.pallas import tpu_sc as plsc

B, L = 4096, 50
VOCAB = 5002
EMBED = 800
LANES = 16
NC, NS = 2, 16
NW = NC * NS
RPW = B // NW
LP64 = 64
VP = 5008
CG = 8


def _counts(x_pad):
    mesh = plsc.VectorSubcoreMesh(
        core_axis_name="c", subcore_axis_name="s", num_cores=NC, num_subcores=NS
    )

    @functools.partial(
        pl.kernel,
        out_type=jax.ShapeDtypeStruct((B, VP), jnp.float32),
        mesh=mesh,
        scratch_types=[
            pltpu.VMEM((RPW, LP64), jnp.int32),
            pltpu.VMEM((CG, VP), jnp.float32),
            pltpu.VMEM((CG, VP), jnp.float32),
            pltpu.SemaphoreType.DMA,
            pltpu.SemaphoreType.DMA,
        ],
        compiler_params=pltpu.CompilerParams(
            use_tc_tiling_on_sc=False, needs_layout_passes=False
        ),
    )
    def k(x_hbm, out_hbm, idx_v, ca, cb, sa, sb):
        wid = lax.axis_index("s") * NC + lax.axis_index("c")
        base = wid * RPW
        pltpu.sync_copy(x_hbm.at[pl.ds(base, RPW)], idx_v)

        zv = jnp.zeros((LANES,), jnp.float32)

        def zero_buf(buf):
            def zrow(rr, c):
                def zcol(d, c2):
                    buf[rr, pl.ds(d * LANES, LANES)] = zv
                    return c2
                return lax.fori_loop(0, VP // LANES, zcol, c, unroll=8)
            lax.fori_loop(0, CG, zrow, 0)

        zero_buf(ca)
        zero_buf(cb)

        def do_group(g, buf, sem):
            @pl.when(g >= 2)
            def _():
                pltpu.make_async_copy(buf, out_hbm.at[pl.ds(base, CG)], sem).wait()

            def row(rr, c0):
                r = g * CG + rr
                rv = jnp.full((LANES,), rr, jnp.int32)

                @pl.when(g >= 2)
                def _():
                    for v in range(LP64 // LANES):
                        old = idx_v[r - 2 * CG, pl.ds(v * LANES, LANES)]
                        plsc.store_scatter(buf, [rv, old], zv)

                for v in range(LP64 // LANES):
                    iv = idx_v[r, pl.ds(v * LANES, LANES)]
                    cnt, last = plsc.scan_count(iv)
                    plsc.addupdate_scatter(buf, [rv, iv], cnt.astype(jnp.float32), mask=last)
                return c0

            lax.fori_loop(0, CG, row, 0)
            pltpu.make_async_copy(buf, out_hbm.at[pl.ds(base + g * CG, CG)], sem).start()

        def gpair(gg, carry):
            do_group(2 * gg, ca, sa)
            do_group(2 * gg + 1, cb, sb)
            return carry

        lax.fori_loop(0, RPW // (2 * CG), gpair, 0)
        pltpu.make_async_copy(ca, out_hbm.at[pl.ds(base, CG)], sa).wait()
        pltpu.make_async_copy(cb, out_hbm.at[pl.ds(base, CG)], sb).wait()

    return k(x_pad)


BB = 256


def _fused_body(c_ref, embt, w1, b1, w2, b2, w3, b3, w4, b4, w5, b5, o):
    c = c_ref[...].astype(jnp.bfloat16)
    h = jnp.dot(c, embt[...], preferred_element_type=jnp.float32) * (1.0 / L)
    h = jnp.maximum(jnp.dot(h.astype(jnp.bfloat16), w1[...], preferred_element_type=jnp.float32) + b1[...], 0.0)
    h = jnp.maximum(jnp.dot(h.astype(jnp.bfloat16), w2[...], preferred_element_type=jnp.float32) + b2[...], 0.0)
    h = jnp.maximum(jnp.dot(h.astype(jnp.bfloat16), w3[...], preferred_element_type=jnp.float32) + b3[...], 0.0)
    h = jnp.maximum(jnp.dot(h.astype(jnp.bfloat16), w4[...], preferred_element_type=jnp.float32) + b4[...], 0.0)
    o[...] = jnp.dot(h.astype(jnp.bfloat16), w5[...], preferred_element_type=jnp.float32) + b5[...]


def _fused_mlp(counts, embt, w1, b1, w2, b2, w3, b3, w4, b4, w5p, b5p):
    full = lambda a: pl.BlockSpec(a.shape, lambda i: (0,) * a.ndim)
    return pl.pallas_call(
        _fused_body,
        grid=(B // BB,),
        in_specs=[pl.BlockSpec((BB, VP), lambda i: (i, 0))]
        + [full(a) for a in (embt, w1, b1, w2, b2, w3, b3, w4, b4, w5p, b5p)],
        out_specs=pl.BlockSpec((BB, 128), lambda i: (i, 0)),
        out_shape=jax.ShapeDtypeStruct((B, 128), jnp.float32),
        compiler_params=pltpu.CompilerParams(dimension_semantics=("arbitrary",)),
    )(counts, embt, w1, b1, w2, b2, w3, b3, w4, b4, w5p, b5p)


def kernel(x, emb, W1, b1, W2, b2, W3, b3, W4, b4, W5, b5):
    x_pad = jnp.concatenate(
        [x, jnp.full((B, LP64 - L), VOCAB, jnp.int32)], axis=1
    )
    counts = _counts(x_pad)

    bf = jnp.bfloat16
    embt = jnp.pad(emb, ((0, VP - VOCAB), (0, 0))).astype(bf)
    w5p = jnp.pad(W5, ((0, 0), (0, 128 - W5.shape[1])))
    b5p = jnp.pad(b5, ((0, 128 - b5.shape[0]),))
    out = _fused_mlp(
        counts, embt,
        W1.astype(bf), b1.reshape(1, -1),
        W2.astype(bf), b2.reshape(1, -1),
        W3.astype(bf), b3.reshape(1, -1),
        W4.astype(bf), b4.reshape(1, -1),
        w5p.astype(bf), b5p.reshape(1, -1),
    )
    return out[:, : W5.shape[1]]


# speedup vs baseline: 5.3724x; 1.0081x over previous
import functools

import jax
import jax.numpy as jnp
from jax import lax
from jax.experimental import pallas as pl
from jax.experimental.pallas import tpu as pltpu
from jax.experimental.pallas import tpu_sc as plsc

B, L = 4096, 50
VOCAB = 5002
EMBED = 800
LANES = 16
NC, NS = 2, 16
NW = NC * NS
RPW = B // NW
LP64 = 64
VP = 5008
CG = 8


def _counts(x_pad):
    mesh = plsc.VectorSubcoreMesh(
        core_axis_name="c", subcore_axis_name="s", num_cores=NC, num_subcores=NS
    )

    @functools.partial(
        pl.kernel,
        out_type=jax.ShapeDtypeStruct((B, VP), jnp.float32),
        mesh=mesh,
        scratch_types=[
            pltpu.VMEM((RPW, LP64), jnp.int32),
            pltpu.VMEM((CG, VP), jnp.float32),
            pltpu.VMEM((CG, VP), jnp.float32),
            pltpu.SemaphoreType.DMA,
            pltpu.SemaphoreType.DMA,
        ],
        compiler_params=pltpu.CompilerParams(
            use_tc_tiling_on_sc=False, needs_layout_passes=False
        ),
    )
    def k(x_hbm, out_hbm, idx_v, ca, cb, sa, sb):
        wid = lax.axis_index("s") * NC + lax.axis_index("c")
        base = wid * RPW
        pltpu.sync_copy(x_hbm.at[pl.ds(base, RPW)], idx_v)

        zv = jnp.zeros((LANES,), jnp.float32)

        def zero_buf(buf):
            def zrow(rr, c):
                def zcol(d, c2):
                    buf[rr, pl.ds(d * LANES, LANES)] = zv
                    return c2
                return lax.fori_loop(0, VP // LANES, zcol, c, unroll=8)
            lax.fori_loop(0, CG, zrow, 0)

        zero_buf(ca)
        zero_buf(cb)

        def do_group(g, buf, sem):
            @pl.when(g >= 2)
            def _():
                pltpu.make_async_copy(buf, out_hbm.at[pl.ds(base, CG)], sem).wait()

            def row(rr, c0):
                r = g * CG + rr
                rv = jnp.full((LANES,), rr, jnp.int32)

                @pl.when(g >= 2)
                def _():
                    for v in range(LP64 // LANES):
                        old = idx_v[r - 2 * CG, pl.ds(v * LANES, LANES)]
                        plsc.store_scatter(buf, [rv, old], zv)

                for v in range(LP64 // LANES):
                    iv = idx_v[r, pl.ds(v * LANES, LANES)]
                    cnt, last = plsc.scan_count(iv)
                    plsc.addupdate_scatter(buf, [rv, iv], cnt.astype(jnp.float32), mask=last)
                return c0

            lax.fori_loop(0, CG, row, 0)
            pltpu.make_async_copy(buf, out_hbm.at[pl.ds(base + g * CG, CG)], sem).start()

        def gpair(gg, carry):
            do_group(2 * gg, ca, sa)
            do_group(2 * gg + 1, cb, sb)
            return carry

        lax.fori_loop(0, RPW // (2 * CG), gpair, 0)
        pltpu.make_async_copy(ca, out_hbm.at[pl.ds(base, CG)], sa).wait()
        pltpu.make_async_copy(cb, out_hbm.at[pl.ds(base, CG)], sb).wait()

    return k(x_pad)


BB = 512


def _fused_body(c_ref, embt, w1, b1, w2, b2, w3, b3, w4, b4, w5, b5, o):
    c = c_ref[...].astype(jnp.bfloat16)
    h = jnp.dot(c, embt[...], preferred_element_type=jnp.float32) * (1.0 / L)
    h = jnp.maximum(jnp.dot(h.astype(jnp.bfloat16), w1[...], preferred_element_type=jnp.float32) + b1[...], 0.0)
    h = jnp.maximum(jnp.dot(h.astype(jnp.bfloat16), w2[...], preferred_element_type=jnp.float32) + b2[...], 0.0)
    h = jnp.maximum(jnp.dot(h.astype(jnp.bfloat16), w3[...], preferred_element_type=jnp.float32) + b3[...], 0.0)
    h = jnp.maximum(jnp.dot(h.astype(jnp.bfloat16), w4[...], preferred_element_type=jnp.float32) + b4[...], 0.0)
    o[...] = jnp.dot(h.astype(jnp.bfloat16), w5[...], preferred_element_type=jnp.float32) + b5[...]


def _fused_mlp(counts, embt, w1, b1, w2, b2, w3, b3, w4, b4, w5p, b5p):
    full = lambda a: pl.BlockSpec(a.shape, lambda i: (0,) * a.ndim)
    return pl.pallas_call(
        _fused_body,
        grid=(B // BB,),
        in_specs=[pl.BlockSpec((BB, VP), lambda i: (i, 0))]
        + [full(a) for a in (embt, w1, b1, w2, b2, w3, b3, w4, b4, w5p, b5p)],
        out_specs=pl.BlockSpec((BB, 128), lambda i: (i, 0)),
        out_shape=jax.ShapeDtypeStruct((B, 128), jnp.float32),
        compiler_params=pltpu.CompilerParams(dimension_semantics=("arbitrary",)),
    )(counts, embt, w1, b1, w2, b2, w3, b3, w4, b4, w5p, b5p)


def kernel(x, emb, W1, b1, W2, b2, W3, b3, W4, b4, W5, b5):
    x_pad = jnp.concatenate(
        [x, jnp.full((B, LP64 - L), VOCAB, jnp.int32)], axis=1
    )
    counts = _counts(x_pad)

    bf = jnp.bfloat16
    embt = jnp.pad(emb, ((0, VP - VOCAB), (0, 0))).astype(bf)
    w5p = jnp.pad(W5, ((0, 0), (0, 128 - W5.shape[1])))
    b5p = jnp.pad(b5, ((0, 128 - b5.shape[0]),))
    out = _fused_mlp(
        counts, embt,
        W1.astype(bf), b1.reshape(1, -1),
        W2.astype(bf), b2.reshape(1, -1),
        W3.astype(bf), b3.reshape(1, -1),
        W4.astype(bf), b4.reshape(1, -1),
        w5p.astype(bf), b5p.reshape(1, -1),
    )
    return out[:, : W5.shape[1]]
